# Initial kernel scaffold; baseline (speedup 1.0000x reference)
import jax
import jax.numpy as jnp
from jax.experimental import pallas as pl


def kernel(batch, edge_index, emb_table, W1, b1, W2, b2):
    raise NotImplementedError("write your pallas kernel here")


# speedup vs baseline: 5.5895x; 5.5895x over previous
import functools

import jax
import jax.numpy as jnp
from jax import lax
from jax.experimental import pallas as pl
from jax.experimental.pallas import tpu as pltpu
from jax.experimental.pallas import tpu_sc as plsc

N_NODES = 10000
NPAD = 10240
VOCAB = 100000
D = 128
E = 320000
CHUNK = 128
N_ECHUNK = E // CHUNK
N_HCHUNK = NPAD // CHUNK
NW = 32
ROWS_PER_TILE = NPAD // 16


def _mesh():
    return plsc.VectorSubcoreMesh(
        core_axis_name="c", subcore_axis_name="s", num_cores=2, num_subcores=16
    )


def _sc_prep_body(batch_hbm, src_hbm, dst_hbm, emb_hbm,
                  h0_hbm, degp_hbm,
                  idx_v, rows_v, hs_v, hd_v, sidx_v, didx_v, sem):
    cid = lax.axis_index("c")
    sid = lax.axis_index("s")
    w = sid * 2 + cid
    z16 = jnp.zeros((16,), jnp.float32)

    def zl(i, carry):
        hs_v[pl.ds(i * 16, 16)] = z16
        hd_v[pl.ds(i * 16, 16)] = z16
        return carry
    lax.fori_loop(0, NPAD // 16, zl, 0)

    def gchunk(j, carry):
        c = w + NW * j

        @pl.when(c < N_HCHUNK)
        def _():
            pltpu.sync_copy(batch_hbm.at[pl.ds(c * CHUNK, CHUNK)], idx_v)
            pltpu.async_copy(emb_hbm.at[idx_v], rows_v, sem).wait()
            pltpu.sync_copy(rows_v, h0_hbm.at[pl.ds(c * CHUNK, CHUNK)])
        return carry
    lax.fori_loop(0, 3, gchunk, 0)

    ones16 = jnp.full((16,), 1.0, jnp.float32)

    def dchunk(j, carry):
        c = w + NW * j

        @pl.when(c < N_ECHUNK)
        def _():
            pltpu.sync_copy(src_hbm.at[pl.ds(c * CHUNK, CHUNK)], sidx_v)
            pltpu.sync_copy(dst_hbm.at[pl.ds(c * CHUNK, CHUNK)], didx_v)
            for jj in range(CHUNK // 16):
                plsc.addupdate_scatter(
                    hs_v, [sidx_v[pl.ds(jj * 16, 16)]], ones16)
                plsc.addupdate_scatter(
                    hd_v, [didx_v[pl.ds(jj * 16, 16)]], ones16)
        return carry
    lax.fori_loop(0, (N_ECHUNK + NW - 1) // NW, dchunk, 0)

    pltpu.sync_copy(hs_v, degp_hbm.at[w, 0])
    pltpu.sync_copy(hd_v, degp_hbm.at[w, 1])


def _sc_prep(batch_pad, src, dst, emb_table):
    f = functools.partial(
        pl.kernel,
        out_type=(
            jax.ShapeDtypeStruct((NPAD, D), jnp.float32),
            jax.ShapeDtypeStruct((NW, 2, NPAD), jnp.float32),
        ),
        mesh=_mesh(),
        scratch_types=[
            pltpu.VMEM((CHUNK,), jnp.int32),
            pltpu.VMEM((CHUNK, D), jnp.float32),
            pltpu.VMEM((NPAD,), jnp.float32),
            pltpu.VMEM((NPAD,), jnp.float32),
            pltpu.VMEM((CHUNK,), jnp.int32),
            pltpu.VMEM((CHUNK,), jnp.int32),
            pltpu.SemaphoreType.DMA,
        ],
        compiler_params=pltpu.CompilerParams(needs_layout_passes=False),
    )(_sc_prep_body)
    return f(batch_pad, src, dst, emb_table)


def _sc_msgpass_body(hw_hbm, src_hbm, dst_hbm, part_hbm,
                     sidx_v, didx_v, rows_v, zbuf_v, agg_sh, sem):
    cid = lax.axis_index("c")
    sid = lax.axis_index("s")
    w = sid * 2 + cid
    z16 = jnp.zeros((16,), jnp.float32)

    for i in range(16):
        for jj in range(D // 16):
            zbuf_v[i, pl.ds(jj * 16, 16)] = z16
    for t in range(ROWS_PER_TILE // 16):
        pltpu.sync_copy(zbuf_v, agg_sh.at[pl.ds(sid * ROWS_PER_TILE + t * 16, 16)])
    plsc.subcore_barrier()

    def ec(j, carry):
        c = w + NW * j

        @pl.when(c < N_ECHUNK)
        def _():
            pltpu.sync_copy(src_hbm.at[pl.ds(c * CHUNK, CHUNK)], sidx_v)
            pltpu.sync_copy(dst_hbm.at[pl.ds(c * CHUNK, CHUNK)], didx_v)
            pltpu.async_copy(hw_hbm.at[sidx_v], rows_v, sem).wait()
            pltpu.sync_copy(rows_v, agg_sh.at[didx_v], add=True)
        return carry
    lax.fori_loop(0, (N_ECHUNK + NW - 1) // NW, ec, 0)

    plsc.subcore_barrier()
    for t in range(ROWS_PER_TILE // CHUNK):
        r0 = sid * ROWS_PER_TILE + t * CHUNK
        pltpu.sync_copy(agg_sh.at[pl.ds(r0, CHUNK)],
                        part_hbm.at[cid, pl.ds(r0, CHUNK)])


def _sc_msgpass(hw, src, dst):
    f = functools.partial(
        pl.kernel,
        out_type=jax.ShapeDtypeStruct((2, NPAD, D), jnp.float32),
        mesh=_mesh(),
        scratch_types=[
            pltpu.VMEM((CHUNK,), jnp.int32),
            pltpu.VMEM((CHUNK,), jnp.int32),
            pltpu.VMEM((CHUNK, D), jnp.float32),
            pltpu.VMEM((16, D), jnp.float32),
            pltpu.VMEM_SHARED((NPAD, D), jnp.float32),
            pltpu.SemaphoreType.DMA,
        ],
        compiler_params=pltpu.CompilerParams(needs_layout_passes=False),
    )(_sc_msgpass_body)
    return f(hw, src, dst)


def _tc_norms_body(degp_ref, norms_ref):
    d = degp_ref[...]
    deg = jnp.sum(d, axis=0)
    norms_ref[...] = jnp.where(
        deg > 0, lax.rsqrt(jnp.maximum(deg, 1.0)), 0.0)


def _tc_norms(degp):
    return pl.pallas_call(
        _tc_norms_body,
        out_shape=jax.ShapeDtypeStruct((2, NPAD), jnp.float32),
    )(degp)


R = 1024
GRID = NPAD // R


def _tc_l1_body(h_ref, w_ref, ns_ref, out_ref):
    hw = jnp.dot(h_ref[...], w_ref[...], preferred_element_type=jnp.float32)
    out_ref[...] = hw * ns_ref[...]


def _tc_l1(h0, W1, ns):
    return pl.pallas_call(
        _tc_l1_body,
        grid=(GRID,),
        in_specs=[
            pl.BlockSpec((R, D), lambda i: (i, 0)),
            pl.BlockSpec((D, D), lambda i: (0, 0)),
            pl.BlockSpec((R, 1), lambda i: (i, 0)),
        ],
        out_specs=pl.BlockSpec((R, D), lambda i: (i, 0)),
        out_shape=jax.ShapeDtypeStruct((NPAD, D), jnp.float32),
    )(h0, W1, ns)


def _tc_mid_body(p_ref, nd_ref, b_ref, w_ref, ns_ref, out_ref):
    agg = p_ref[0] + p_ref[1]
    h = jnp.maximum(agg * nd_ref[...] + b_ref[...], 0.0)
    out_ref[...] = jnp.dot(
        h, w_ref[...], preferred_element_type=jnp.float32) * ns_ref[...]


def _tc_mid(p, nd, b1, W2, ns):
    return pl.pallas_call(
        _tc_mid_body,
        grid=(GRID,),
        in_specs=[
            pl.BlockSpec((2, R, D), lambda i: (0, i, 0)),
            pl.BlockSpec((R, 1), lambda i: (i, 0)),
            pl.BlockSpec((1, D), lambda i: (0, 0)),
            pl.BlockSpec((D, D), lambda i: (0, 0)),
            pl.BlockSpec((R, 1), lambda i: (i, 0)),
        ],
        out_specs=pl.BlockSpec((R, D), lambda i: (i, 0)),
        out_shape=jax.ShapeDtypeStruct((NPAD, D), jnp.float32),
    )(p, nd, b1, W2, ns)


def _tc_fin_body(q_ref, nd_ref, b_ref, out_ref):
    agg = q_ref[0] + q_ref[1]
    out_ref[...] = jnp.maximum(agg * nd_ref[...] + b_ref[...], 0.0)


def _tc_fin(q, nd, b2):
    return pl.pallas_call(
        _tc_fin_body,
        grid=(GRID,),
        in_specs=[
            pl.BlockSpec((2, R, D), lambda i: (0, i, 0)),
            pl.BlockSpec((R, 1), lambda i: (i, 0)),
            pl.BlockSpec((1, D), lambda i: (0, 0)),
        ],
        out_specs=pl.BlockSpec((R, D), lambda i: (i, 0)),
        out_shape=jax.ShapeDtypeStruct((NPAD, D), jnp.float32),
    )(q, nd, b2)


def kernel(batch, edge_index, emb_table, W1, b1, W2, b2):
    src = edge_index[0].astype(jnp.int32)
    dst = edge_index[1].astype(jnp.int32)
    batch_pad = jnp.concatenate(
        [batch.astype(jnp.int32), jnp.zeros((NPAD - N_NODES,), jnp.int32)])

    h0, degp = _sc_prep(batch_pad, src, dst, emb_table)
    norms = _tc_norms(degp)
    ns = norms[0].reshape(NPAD, 1)
    nd = norms[1].reshape(NPAD, 1)

    hw1 = _tc_l1(h0, W1, ns)
    p1 = _sc_msgpass(hw1, src, dst)
    hw2 = _tc_mid(p1, nd, b1.reshape(1, D), W2, ns)
    p2 = _sc_msgpass(hw2, src, dst)
    out = _tc_fin(p2, nd, b2.reshape(1, D))
    return out[:N_NODES]
